# baseline (speedup 1.0000x reference)
import functools

import jax
import jax.numpy as jnp
from jax import lax
from jax.experimental import pallas as pl
from jax.experimental.pallas import tpu as pltpu
from jax.experimental.pallas import tpu_sc as plsc

N = 10000
E = 160000
D = 128

L = 16
NC = 2
NS = 16
NW = NC * NS
B = 128
NCHUNK = E // B
JMAX = (NCHUNK + NW - 1) // NW
BN = 64
NCHUNKN = E // BN
JMAXN = (NCHUNKN + NW - 1) // NW
ROWS_T = 624
REM = N - NS * ROWS_T
ZROWS = 128

_f32 = jnp.float32


@functools.cache
def _mesh():
    return plsc.VectorSubcoreMesh(core_axis_name="c", subcore_axis_name="s",
                                  num_cores=NC)


def _zero_vmem_rows(buf, rows, width):
    def body(i, _):
        for l in range(width // L):
            buf[i, pl.ds(l * L, L)] = jnp.zeros((L,), _f32)
        return 0
    lax.fori_loop(0, rows, body, 0)


RCH = 16
BN2 = 40
NCW = E // NW // BN2


@functools.cache
def _make_nc_sc(with_deg):
    dbl = not with_deg
    out_type = [jax.ShapeDtypeStruct((NC, N, D), _f32)]
    scratch = [
        pltpu.VMEM((NW * NCW // NW * BN2,), jnp.int32),
        pltpu.VMEM((NCW, BN2), jnp.int32),
        pltpu.VMEM((BN2, D), _f32),
        pltpu.VMEM((BN2, D), _f32),
        pltpu.VMEM((D,), _f32),
        pltpu.VMEM((RCH,), jnp.int32),
        pltpu.VMEM((RCH, D), _f32),
        pltpu.VMEM_SHARED((N, D), _f32),
        pltpu.SemaphoreType.DMA,
        pltpu.SemaphoreType.DMA,
    ]
    if dbl:
        scratch += [
            pltpu.VMEM((BN2, D), _f32),
            pltpu.VMEM((BN2, D), _f32),
            pltpu.SemaphoreType.DMA,
            pltpu.SemaphoreType.DMA,
        ]
    if with_deg:
        out_type.append(jax.ShapeDtypeStruct((NC, N, L), _f32))
        scratch += [
            pltpu.VMEM((BN2, L), _f32),
            pltpu.VMEM((RCH, L), _f32),
            pltpu.VMEM_SHARED((N, L), _f32),
            pltpu.SemaphoreType.DMA,
        ]

    @functools.partial(
        pl.kernel, mesh=_mesh(), out_type=out_type, scratch_types=scratch,
        compiler_params=pltpu.CompilerParams(use_tc_tiling_on_sc=False))
    def nc_kernel(src_hbm, dst2_hbm, pt_hbm, pb_hbm, b0_hbm, rows_hbm,
                  *rest):
        if with_deg:
            (s_out, deg_out, sidx, didx2, a0, g0, b0buf, ridx,
             stage, s_sh, ss0, sd0,
             ones_v, dstage, deg_sh, sdg) = rest
            sets = ((a0, g0, ss0, sd0),)
        else:
            (s_out, sidx, didx2, a0, g0, b0buf, ridx,
             stage, s_sh, ss0, sd0, a1, g1, ss1, sd1) = rest
            sets = ((a0, g0, ss0, sd0), (a1, g1, ss1, sd1))
        cid = lax.axis_index("c")
        sid = lax.axis_index("s")
        wid = sid * NC + cid
        tbase = sid * ROWS_T
        nedge = NCW * BN2

        def load_ridx(base):
            pltpu.sync_copy(rows_hbm.at[pl.ds(base, RCH)], ridx)

        pltpu.sync_copy(src_hbm.at[pl.ds(wid * nedge, nedge)], sidx)
        pltpu.sync_copy(dst2_hbm.at[pl.ds(wid * NCW, NCW)], didx2)

        _zero_vmem_rows(stage, RCH, D)
        if with_deg:
            _zero_vmem_rows(dstage, RCH, L)

            def ones_body(i, _):
                ones_v[i, :] = jnp.ones((L,), _f32)
                return 0
            lax.fori_loop(0, BN2, ones_body, 0)

        pltpu.sync_copy(b0_hbm, b0buf)
        b0v = [b0buf[pl.ds(l * L, L)] for l in range(D // L)]

        for k in range(ROWS_T // RCH):
            load_ridx(tbase + k * RCH)
            pltpu.sync_copy(stage, s_sh.at[ridx])
            if with_deg:
                pltpu.sync_copy(dstage, deg_sh.at[ridx])

        @pl.when(sid == 0)
        def _():
            load_ridx(NS * ROWS_T)
            pltpu.sync_copy(stage, s_sh.at[ridx])
            if with_deg:
                pltpu.sync_copy(dstage, deg_sh.at[ridx])
        plsc.subcore_barrier()

        def fire(k, s):
            a_, g_, sa_, sg_ = sets[s]
            pltpu.async_copy(pt_hbm.at[sidx.at[pl.ds(k * BN2, BN2)]], a_, sa_)
            pltpu.async_copy(pb_hbm.at[didx2.at[k]], g_, sg_)

        def finish(k, s):
            a_, g_, sa_, sg_ = sets[s]
            pltpu.make_async_copy(
                pt_hbm.at[sidx.at[pl.ds(k * BN2, BN2)]], a_, sa_).wait()
            pltpu.make_async_copy(pb_hbm.at[didx2.at[k]], g_, sg_).wait()

            def rbody(i, _):
                for l in range(D // L):
                    sl = pl.ds(l * L, L)
                    a_[i, sl] = jnp.maximum(a_[i, sl] + g_[i, sl] + b0v[l],
                                            0.0)
                return 0
            lax.fori_loop(0, BN2, rbody, 0)
            pltpu.sync_copy(a_, s_sh.at[didx2.at[k]], add=True)
            if with_deg:
                @pl.when(k > 0)
                def _():
                    pltpu.make_async_copy(
                        ones_v, deg_sh.at[didx2.at[k]], sdg).wait()
                pltpu.async_copy(ones_v, deg_sh.at[didx2.at[k]], sdg,
                                 add=True)

        if dbl:
            fire(0, 0)

            def body(t, _):
                k0 = 2 * t
                k1 = k0 + 1

                @pl.when(k1 < NCW)
                def _():
                    fire(k1, 1)
                finish(k0, 0)

                @pl.when(k0 + 2 < NCW)
                def _():
                    fire(k0 + 2, 0)

                @pl.when(k1 < NCW)
                def _():
                    finish(k1, 1)
                return 0
            lax.fori_loop(0, (NCW + 1) // 2, body, 0)
        else:
            def body(k, _):
                fire(k, 0)
                finish(k, 0)
                return 0
            lax.fori_loop(0, NCW, body, 0)
        if with_deg:
            pltpu.make_async_copy(ones_v, deg_sh.at[didx2.at[0]],
                                  sdg).wait()
        plsc.subcore_barrier()

        for k in range(ROWS_T // RCH):
            base = tbase + k * RCH
            load_ridx(base)
            pltpu.sync_copy(s_sh.at[ridx], stage)
            pltpu.sync_copy(stage, s_out.at[cid, pl.ds(base, RCH)])
            if with_deg:
                pltpu.sync_copy(deg_sh.at[ridx], dstage)
                pltpu.sync_copy(dstage, deg_out.at[cid, pl.ds(base, RCH)])

        @pl.when(sid == 0)
        def _():
            load_ridx(NS * ROWS_T)
            pltpu.sync_copy(s_sh.at[ridx], stage)
            pltpu.sync_copy(stage, s_out.at[cid, pl.ds(NS * ROWS_T, REM)])
            if with_deg:
                pltpu.sync_copy(deg_sh.at[ridx], dstage)
                pltpu.sync_copy(dstage,
                                deg_out.at[cid, pl.ds(NS * ROWS_T, REM)])

    return nc_kernel


EB = 40
EW = E // NW
ECHUNKS = EW // EB


@functools.cache
def _make_ec_sc():
    @functools.partial(
        pl.kernel, mesh=_mesh(),
        out_type=[jax.ShapeDtypeStruct((E, D), _f32),
                  jax.ShapeDtypeStruct((NW, 8, L), _f32)],
        scratch_types=[
            pltpu.VMEM((EW,), jnp.int32),
            pltpu.VMEM((EW,), jnp.int32),
            pltpu.VMEM((EB, 2 * D), _f32),
            pltpu.VMEM((EB, 2 * D), _f32),
            pltpu.VMEM((EB, 2 * D), _f32),
            pltpu.VMEM((EB, 2 * D), _f32),
            pltpu.VMEM((5 * EB, D), _f32),
            pltpu.VMEM((D,), _f32),
            pltpu.VMEM((8, L), _f32),
            pltpu.SemaphoreType.DMA,
            pltpu.SemaphoreType.DMA,
            pltpu.SemaphoreType.DMA,
            pltpu.SemaphoreType.DMA,
        ],
        compiler_params=pltpu.CompilerParams(use_tc_tiling_on_sc=False))
    def ec_kernel(src_hbm, dst_hbm, p_hbm, b0_hbm, h_out, ssq_out,
                  sidx, didx, ps0, pd0, ps1, pd1, hbuf, b0buf, ssqacc,
                  ss0, sd0, ss1, sd1):
        cid = lax.axis_index("c")
        sid = lax.axis_index("s")
        wid = sid * NC + cid
        ebase0 = wid * EW

        pltpu.sync_copy(src_hbm.at[pl.ds(ebase0, EW)], sidx)
        pltpu.sync_copy(dst_hbm.at[pl.ds(ebase0, EW)], didx)
        pltpu.sync_copy(b0_hbm, b0buf)
        b0v = [b0buf[pl.ds(l * L, L)] for l in range(D // L)]
        for r in range(8):
            ssqacc[r, :] = jnp.zeros((L,), _f32)

        sets = ((ps0, pd0, ss0, sd0), (ps1, pd1, ss1, sd1))

        def fire(k, s):
            ps_, pd_, ss_, sd_ = sets[s]
            pltpu.async_copy(p_hbm.at[sidx.at[pl.ds(k * EB, EB)]], ps_, ss_)
            pltpu.async_copy(p_hbm.at[didx.at[pl.ds(k * EB, EB)]], pd_, sd_)

        def finish(k, s):
            ps_, pd_, ss_, sd_ = sets[s]
            pltpu.make_async_copy(
                p_hbm.at[sidx.at[pl.ds(k * EB, EB)]], ps_, ss_).wait()
            pltpu.make_async_copy(
                p_hbm.at[didx.at[pl.ds(k * EB, EB)]], pd_, sd_).wait()

            hb = lax.rem(k, 5) * EB

            def row(i, racc):
                for l in range(D // L):
                    sl = pl.ds(l * L, L)
                    sl2 = pl.ds(D + l * L, L)
                    a = ps_[i, sl] + pd_[i, sl2] + b0v[l]
                    b = pd_[i, sl] + ps_[i, sl2] + b0v[l]
                    f1 = jnp.maximum(a, 0.0)
                    f2 = jnp.maximum(b, 0.0)
                    hbuf[hb + i, sl] = 0.5 * (f1 + f2)
                    dd = f1 - f2
                    racc = racc + dd * dd
                return racc
            racc = lax.fori_loop(0, EB, row, jnp.zeros((L,), _f32))
            ssqacc[0, :] = ssqacc[0, :] + racc

            @pl.when(lax.rem(k, 5) == 4)
            def _():
                pltpu.sync_copy(
                    hbuf, h_out.at[pl.ds(ebase0 + (k - 4) * EB, 5 * EB)])

        fire(0, 0)

        def body(t, _):
            k0 = 2 * t
            k1 = k0 + 1

            @pl.when(k1 < ECHUNKS)
            def _():
                fire(k1, 1)
            finish(k0, 0)

            @pl.when(k0 + 2 < ECHUNKS)
            def _():
                fire(k0 + 2, 0)

            @pl.when(k1 < ECHUNKS)
            def _():
                finish(k1, 1)
            return 0
        lax.fori_loop(0, (ECHUNKS + 1) // 2, body, 0)
        pltpu.sync_copy(ssqacc, ssq_out.at[wid])

    return ec_kernel


def _node_tc_body(with_update, nproj, refs):
    if with_update:
        x_ref, sp_ref, degp_ref, w1_ref, b1_ref = refs[:5]
        wrefs = refs[5:5 + nproj]
        outs = refs[5 + nproj:]
        xout = outs[0]
        pouts = outs[1:]
        s = sp_ref[0] + sp_ref[1]
        deg = degp_ref[0][:, 0:1] + degp_ref[1][:, 0:1]
        agg = jnp.dot(s / jnp.maximum(deg, 1.0), w1_ref[...],
                      preferred_element_type=_f32)
        agg = agg + jnp.minimum(deg, 1.0) * b1_ref[...]
        xn = jnp.maximum(x_ref[...] + agg, 0.0)
        xout[...] = xn
    else:
        x_ref = refs[0]
        wrefs = refs[1:1 + nproj]
        pouts = refs[1 + nproj:]
        xn = x_ref[...]
    for w_ref, p_ref in zip(wrefs, pouts):
        p_ref[...] = jnp.dot(xn, w_ref[...], preferred_element_type=_f32)


def _make_node_tc(with_update, proj_widths):
    nproj = len(proj_widths)
    out_shape = []
    if with_update:
        out_shape.append(jax.ShapeDtypeStruct((N, D), _f32))
    out_shape += [jax.ShapeDtypeStruct((N, w), _f32) for w in proj_widths]

    def body(*refs):
        _node_tc_body(with_update, nproj, refs)

    return pl.pallas_call(body, out_shape=out_shape)


_node_tc_a = _make_node_tc(False, [D, D])
_node_tc_b = _make_node_tc(True, [2 * D, D, D])
_node_tc_c = _make_node_tc(True, [2 * D])

_BE = 2000


def _ef2_body(h2_ref, h1_ref, act_ref, ang_ref, w2a_ref, w2b_ref,
              w1h_ref, w1f_ref, b1_ref, b2_ref, out_ref):
    w2b = w2b_ref[...]
    f = jnp.dot(w1h_ref[...], w2b, preferred_element_type=_f32)
    f01 = jnp.dot(w1f_ref[...], w2b, preferred_element_type=_f32)
    fb = jnp.dot(b1_ref[...], w2b, preferred_element_type=_f32)
    out_ref[...] = (jnp.dot(h2_ref[...], w2a_ref[...],
                            preferred_element_type=_f32)
                    + jnp.dot(h1_ref[...], f, preferred_element_type=_f32)
                    + act_ref[...] * f01[0:1, :]
                    + ang_ref[...] * f01[1:2, :]
                    + fb + b2_ref[...])


_ef2_asm = pl.pallas_call(
    _ef2_body,
    grid=(E // _BE,),
    in_specs=[
        pl.BlockSpec((_BE, D), lambda i: (i, 0)),
        pl.BlockSpec((_BE, D), lambda i: (i, 0)),
        pl.BlockSpec((_BE, 1), lambda i: (i, 0)),
        pl.BlockSpec((_BE, 1), lambda i: (i, 0)),
        pl.BlockSpec((D, D), lambda i: (0, 0)),
        pl.BlockSpec((2 * D, D), lambda i: (0, 0)),
        pl.BlockSpec((D, 2 * D), lambda i: (0, 0)),
        pl.BlockSpec((2, 2 * D), lambda i: (0, 0)),
        pl.BlockSpec((1, 2 * D), lambda i: (0, 0)),
        pl.BlockSpec((1, D), lambda i: (0, 0)),
    ],
    out_specs=pl.BlockSpec((_BE, D), lambda i: (i, 0)),
    out_shape=jax.ShapeDtypeStruct((E, D), _f32),
)


def _ec_jnp(src, dst, p, b0):
    a = p[src][:, :D] + p[dst][:, D:] + b0
    b = p[dst][:, :D] + p[src][:, D:] + b0
    f1 = jax.nn.relu(a)
    f2 = jax.nn.relu(b)
    return 0.5 * (f1 + f2), jnp.sum((f1 - f2) ** 2)


def kernel(node_features, edge_index, angles, gt_edges, actions,
           nc1_W0, nc1_b0, nc1_W1, nc1_b1,
           nc2_W0, nc2_b0, nc2_W1, nc2_b1,
           ec1_W0, ec1_b0, ec1_W1, ec1_b1,
           ec2_W0, ec2_b0, ec2_W1, ec2_b1):
    src = edge_index[0]
    dst = edge_index[1]
    x0 = node_features

    _nc_sc_deg = _make_nc_sc(True)
    _nc_sc = _make_nc_sc(False)
    _ec_sc = _make_ec_sc()

    pt1, pb1 = _node_tc_a(x0, nc1_W0[:D], nc1_W0[D:])
    rows = jnp.arange(N, dtype=jnp.int32)
    dst2d = dst.reshape(E // BN2, BN2)
    s1, degp = _nc_sc_deg(src, dst2d, pt1, pb1, nc1_b0, rows)
    ec1_wc = jnp.concatenate([ec1_W0[:D], ec1_W0[D:]], axis=1)
    x1, p1, pt2, pb2 = _node_tc_b(x0, s1, degp, nc1_W1,
                                  nc1_b1.reshape(1, D), ec1_wc,
                                  nc2_W0[:D], nc2_W0[D:])

    h1, ssq1 = _ec_sc(src, dst, p1, ec1_b0)
    (s2,) = _nc_sc(src, dst2d, pt2, pb2, nc2_b0, rows)
    ec2_wc = jnp.concatenate([ec2_W0[:D], ec2_W0[D:]], axis=1)
    x2, p2 = _node_tc_c(x1, s2, degp, nc2_W1, nc2_b1.reshape(1, D), ec2_wc)

    h2, ssq2 = _ec_sc(src, dst, p2, ec2_b0)
    ef2 = _ef2_asm(h2, h1, actions, angles, ec2_W1[:D], ec2_W1[D:],
                   ec1_W1[:D], ec1_W1[D:], ec1_b1.reshape(1, 2 * D),
                   ec2_b1.reshape(1, D))

    side_loss = (jnp.sum(ssq1) + jnp.sum(ssq2)) / (2.0 * E * D)
    return ef2, side_loss


# speedup vs baseline: 1.0003x; 1.0003x over previous
import functools

import jax
import jax.numpy as jnp
from jax import lax
from jax.experimental import pallas as pl
from jax.experimental.pallas import tpu as pltpu
from jax.experimental.pallas import tpu_sc as plsc

N = 10000
E = 160000
D = 128

L = 16
NC = 2
NS = 16
NW = NC * NS
B = 128
NCHUNK = E // B
JMAX = (NCHUNK + NW - 1) // NW
BN = 64
NCHUNKN = E // BN
JMAXN = (NCHUNKN + NW - 1) // NW
ROWS_T = 624
REM = N - NS * ROWS_T
ZROWS = 128

_f32 = jnp.float32


@functools.cache
def _mesh():
    return plsc.VectorSubcoreMesh(core_axis_name="c", subcore_axis_name="s",
                                  num_cores=NC)


def _zero_vmem_rows(buf, rows, width):
    def body(i, _):
        for l in range(width // L):
            buf[i, pl.ds(l * L, L)] = jnp.zeros((L,), _f32)
        return 0
    lax.fori_loop(0, rows, body, 0)


RCH = 16
BN2 = 40
NCW = E // NW // BN2


@functools.cache
def _make_nc_sc(with_deg):
    dbl = not with_deg
    out_type = [jax.ShapeDtypeStruct((NC, N, D), _f32)]
    scratch = [
        pltpu.VMEM((NW * NCW // NW * BN2,), jnp.int32),
        pltpu.VMEM((NCW, BN2), jnp.int32),
        pltpu.VMEM((BN2, D), _f32),
        pltpu.VMEM((BN2, D), _f32),
        pltpu.VMEM((D,), _f32),
        pltpu.VMEM((RCH,), jnp.int32),
        pltpu.VMEM((RCH, D), _f32),
        pltpu.VMEM_SHARED((N, D), _f32),
        pltpu.SemaphoreType.DMA,
        pltpu.SemaphoreType.DMA,
    ]
    if dbl:
        scratch += [
            pltpu.VMEM((BN2, D), _f32),
            pltpu.VMEM((BN2, D), _f32),
            pltpu.SemaphoreType.DMA,
            pltpu.SemaphoreType.DMA,
        ]
    if with_deg:
        out_type.append(jax.ShapeDtypeStruct((NC, N, L), _f32))
        scratch += [
            pltpu.VMEM((BN2, L), _f32),
            pltpu.VMEM((RCH, L), _f32),
            pltpu.VMEM_SHARED((N, L), _f32),
            pltpu.SemaphoreType.DMA,
        ]

    @functools.partial(
        pl.kernel, mesh=_mesh(), out_type=out_type, scratch_types=scratch,
        compiler_params=pltpu.CompilerParams(use_tc_tiling_on_sc=False))
    def nc_kernel(src_hbm, dst2_hbm, pt_hbm, pb_hbm, b0_hbm, rows_hbm,
                  *rest):
        if with_deg:
            (s_out, deg_out, sidx, didx2, a0, g0, b0buf, ridx,
             stage, s_sh, ss0, sd0,
             ones_v, dstage, deg_sh, sdg) = rest
            sets = ((a0, g0, ss0, sd0),)
        else:
            (s_out, sidx, didx2, a0, g0, b0buf, ridx,
             stage, s_sh, ss0, sd0, a1, g1, ss1, sd1) = rest
            sets = ((a0, g0, ss0, sd0), (a1, g1, ss1, sd1))
        cid = lax.axis_index("c")
        sid = lax.axis_index("s")
        wid = sid * NC + cid
        tbase = sid * ROWS_T
        nedge = NCW * BN2

        def load_ridx(base):
            pltpu.sync_copy(rows_hbm.at[pl.ds(base, RCH)], ridx)

        pltpu.sync_copy(src_hbm.at[pl.ds(wid * nedge, nedge)], sidx)
        pltpu.sync_copy(dst2_hbm.at[pl.ds(wid * NCW, NCW)], didx2)

        _zero_vmem_rows(stage, RCH, D)
        if with_deg:
            _zero_vmem_rows(dstage, RCH, L)

            def ones_body(i, _):
                ones_v[i, :] = jnp.ones((L,), _f32)
                return 0
            lax.fori_loop(0, BN2, ones_body, 0)

        pltpu.sync_copy(b0_hbm, b0buf)
        b0v = [b0buf[pl.ds(l * L, L)] for l in range(D // L)]

        for k in range(ROWS_T // RCH):
            load_ridx(tbase + k * RCH)
            pltpu.sync_copy(stage, s_sh.at[ridx])
            if with_deg:
                pltpu.sync_copy(dstage, deg_sh.at[ridx])

        @pl.when(sid == 0)
        def _():
            load_ridx(NS * ROWS_T)
            pltpu.sync_copy(stage, s_sh.at[ridx])
            if with_deg:
                pltpu.sync_copy(dstage, deg_sh.at[ridx])
        plsc.subcore_barrier()

        def fire(k, s):
            a_, g_, sa_, sg_ = sets[s]
            pltpu.async_copy(pt_hbm.at[sidx.at[pl.ds(k * BN2, BN2)]], a_, sa_)
            pltpu.async_copy(pb_hbm.at[didx2.at[k]], g_, sg_)

        def finish(k, s):
            a_, g_, sa_, sg_ = sets[s]
            pltpu.make_async_copy(
                pt_hbm.at[sidx.at[pl.ds(k * BN2, BN2)]], a_, sa_).wait()
            pltpu.make_async_copy(pb_hbm.at[didx2.at[k]], g_, sg_).wait()

            def rbody(i, _):
                for l in range(D // L):
                    sl = pl.ds(l * L, L)
                    a_[i, sl] = jnp.maximum(a_[i, sl] + g_[i, sl] + b0v[l],
                                            0.0)
                return 0
            lax.fori_loop(0, BN2, rbody, 0)
            pltpu.sync_copy(a_, s_sh.at[didx2.at[k]], add=True)
            if with_deg:
                @pl.when(k > 0)
                def _():
                    pltpu.make_async_copy(
                        ones_v, deg_sh.at[didx2.at[k]], sdg).wait()
                pltpu.async_copy(ones_v, deg_sh.at[didx2.at[k]], sdg,
                                 add=True)

        if dbl:
            fire(0, 0)

            def body(t, _):
                k0 = 2 * t
                k1 = k0 + 1

                @pl.when(k1 < NCW)
                def _():
                    fire(k1, 1)
                finish(k0, 0)

                @pl.when(k0 + 2 < NCW)
                def _():
                    fire(k0 + 2, 0)

                @pl.when(k1 < NCW)
                def _():
                    finish(k1, 1)
                return 0
            lax.fori_loop(0, (NCW + 1) // 2, body, 0)
        else:
            def body(k, _):
                fire(k, 0)
                finish(k, 0)
                return 0
            lax.fori_loop(0, NCW, body, 0)
        if with_deg:
            pltpu.make_async_copy(ones_v, deg_sh.at[didx2.at[0]],
                                  sdg).wait()
        plsc.subcore_barrier()

        for k in range(ROWS_T // RCH):
            base = tbase + k * RCH
            load_ridx(base)
            pltpu.sync_copy(s_sh.at[ridx], stage)
            pltpu.sync_copy(stage, s_out.at[cid, pl.ds(base, RCH)])
            if with_deg:
                pltpu.sync_copy(deg_sh.at[ridx], dstage)
                pltpu.sync_copy(dstage, deg_out.at[cid, pl.ds(base, RCH)])

        @pl.when(sid == 0)
        def _():
            load_ridx(NS * ROWS_T)
            pltpu.sync_copy(s_sh.at[ridx], stage)
            pltpu.sync_copy(stage, s_out.at[cid, pl.ds(NS * ROWS_T, REM)])
            if with_deg:
                pltpu.sync_copy(deg_sh.at[ridx], dstage)
                pltpu.sync_copy(dstage,
                                deg_out.at[cid, pl.ds(NS * ROWS_T, REM)])

    return nc_kernel


EB = 40
EW = E // NW
ECHUNKS = EW // EB


@functools.cache
def _make_ec_sc():
    @functools.partial(
        pl.kernel, mesh=_mesh(),
        out_type=[jax.ShapeDtypeStruct((E, D), _f32),
                  jax.ShapeDtypeStruct((NW, 8, L), _f32)],
        scratch_types=[
            pltpu.VMEM((EW,), jnp.int32),
            pltpu.VMEM((EW,), jnp.int32),
            pltpu.VMEM((EB, 2 * D), _f32),
            pltpu.VMEM((EB, 2 * D), _f32),
            pltpu.VMEM((EB, 2 * D), _f32),
            pltpu.VMEM((EB, 2 * D), _f32),
            pltpu.VMEM((5 * EB, D), _f32),
            pltpu.VMEM((D,), _f32),
            pltpu.VMEM((8, L), _f32),
            pltpu.SemaphoreType.DMA,
            pltpu.SemaphoreType.DMA,
            pltpu.SemaphoreType.DMA,
            pltpu.SemaphoreType.DMA,
        ],
        compiler_params=pltpu.CompilerParams(use_tc_tiling_on_sc=False))
    def ec_kernel(src_hbm, dst_hbm, p_hbm, b0_hbm, h_out, ssq_out,
                  sidx, didx, ps0, pd0, ps1, pd1, hbuf, b0buf, ssqacc,
                  ss0, sd0, ss1, sd1):
        cid = lax.axis_index("c")
        sid = lax.axis_index("s")
        wid = sid * NC + cid
        ebase0 = wid * EW

        pltpu.sync_copy(src_hbm.at[pl.ds(ebase0, EW)], sidx)
        pltpu.sync_copy(dst_hbm.at[pl.ds(ebase0, EW)], didx)
        pltpu.sync_copy(b0_hbm, b0buf)
        b0v = [b0buf[pl.ds(l * L, L)] for l in range(D // L)]
        for r in range(8):
            ssqacc[r, :] = jnp.zeros((L,), _f32)

        sets = ((ps0, pd0, ss0, sd0), (ps1, pd1, ss1, sd1))

        def fire(k, s):
            ps_, pd_, ss_, sd_ = sets[s]
            pltpu.async_copy(p_hbm.at[sidx.at[pl.ds(k * EB, EB)]], ps_, ss_)
            pltpu.async_copy(p_hbm.at[didx.at[pl.ds(k * EB, EB)]], pd_, sd_)

        def finish(k, s):
            ps_, pd_, ss_, sd_ = sets[s]
            pltpu.make_async_copy(
                p_hbm.at[sidx.at[pl.ds(k * EB, EB)]], ps_, ss_).wait()
            pltpu.make_async_copy(
                p_hbm.at[didx.at[pl.ds(k * EB, EB)]], pd_, sd_).wait()

            hb = lax.rem(k, 5) * EB

            def row(i, racc):
                for l in range(D // L):
                    sl = pl.ds(l * L, L)
                    sl2 = pl.ds(D + l * L, L)
                    a = ps_[i, sl] + pd_[i, sl2] + b0v[l]
                    b = pd_[i, sl] + ps_[i, sl2] + b0v[l]
                    f1 = jnp.maximum(a, 0.0)
                    f2 = jnp.maximum(b, 0.0)
                    hbuf[hb + i, sl] = 0.5 * (f1 + f2)
                    dd = f1 - f2
                    racc = racc + dd * dd
                return racc
            racc = lax.fori_loop(0, EB, row, jnp.zeros((L,), _f32))
            ssqacc[0, :] = ssqacc[0, :] + racc

            @pl.when(lax.rem(k, 5) == 4)
            def _():
                pltpu.sync_copy(
                    hbuf, h_out.at[pl.ds(ebase0 + (k - 4) * EB, 5 * EB)])

        fire(0, 0)

        def body(t, _):
            k0 = 2 * t
            k1 = k0 + 1

            @pl.when(k1 < ECHUNKS)
            def _():
                fire(k1, 1)
            finish(k0, 0)

            @pl.when(k0 + 2 < ECHUNKS)
            def _():
                fire(k0 + 2, 0)

            @pl.when(k1 < ECHUNKS)
            def _():
                finish(k1, 1)
            return 0
        lax.fori_loop(0, (ECHUNKS + 1) // 2, body, 0)
        pltpu.sync_copy(ssqacc, ssq_out.at[wid])

    return ec_kernel


def _node_tc_body(with_update, nproj, refs):
    if with_update:
        x_ref, sp_ref, degp_ref, w1_ref, b1_ref = refs[:5]
        wrefs = refs[5:5 + nproj]
        outs = refs[5 + nproj:]
        xout = outs[0]
        pouts = outs[1:]
        s = sp_ref[0] + sp_ref[1]
        deg = degp_ref[0][:, 0:1] + degp_ref[1][:, 0:1]
        agg = jnp.dot(s / jnp.maximum(deg, 1.0), w1_ref[...],
                      preferred_element_type=_f32)
        agg = agg + jnp.minimum(deg, 1.0) * b1_ref[...]
        xn = jnp.maximum(x_ref[...] + agg, 0.0)
        xout[...] = xn
    else:
        x_ref = refs[0]
        wrefs = refs[1:1 + nproj]
        pouts = refs[1 + nproj:]
        xn = x_ref[...]
    for w_ref, p_ref in zip(wrefs, pouts):
        p_ref[...] = jnp.dot(xn, w_ref[...], preferred_element_type=_f32)


def _make_node_tc(with_update, proj_widths):
    nproj = len(proj_widths)
    out_shape = []
    if with_update:
        out_shape.append(jax.ShapeDtypeStruct((N, D), _f32))
    out_shape += [jax.ShapeDtypeStruct((N, w), _f32) for w in proj_widths]

    def body(*refs):
        _node_tc_body(with_update, nproj, refs)

    return pl.pallas_call(body, out_shape=out_shape)


_node_tc_a = _make_node_tc(False, [D, D])
_node_tc_b = _make_node_tc(True, [2 * D, D, D])
_node_tc_c = _make_node_tc(True, [2 * D])

_BE = 2000


def _fold_body(w1h_ref, w1f_ref, b1_ref, w2b_ref, f_ref, f01_ref, fb_ref):
    w2b = w2b_ref[...]
    f_ref[...] = jnp.dot(w1h_ref[...], w2b, preferred_element_type=_f32)
    f01_ref[...] = jnp.dot(w1f_ref[...], w2b, preferred_element_type=_f32)
    fb_ref[...] = jnp.dot(b1_ref[...], w2b, preferred_element_type=_f32)


_fold_w = pl.pallas_call(
    _fold_body,
    out_shape=[jax.ShapeDtypeStruct((D, D), _f32),
               jax.ShapeDtypeStruct((2, D), _f32),
               jax.ShapeDtypeStruct((1, D), _f32)],
)


def _ef2_body(h2_ref, h1_ref, act_ref, ang_ref, w2a_ref, f_ref,
              f01_ref, fb_ref, b2_ref, out_ref):
    out_ref[...] = (jnp.dot(h2_ref[...], w2a_ref[...],
                            preferred_element_type=_f32)
                    + jnp.dot(h1_ref[...], f_ref[...],
                              preferred_element_type=_f32)
                    + act_ref[...] * f01_ref[0:1, :]
                    + ang_ref[...] * f01_ref[1:2, :]
                    + fb_ref[...] + b2_ref[...])


_ef2_asm = pl.pallas_call(
    _ef2_body,
    grid=(E // _BE,),
    in_specs=[
        pl.BlockSpec((_BE, D), lambda i: (i, 0)),
        pl.BlockSpec((_BE, D), lambda i: (i, 0)),
        pl.BlockSpec((_BE, 1), lambda i: (i, 0)),
        pl.BlockSpec((_BE, 1), lambda i: (i, 0)),
        pl.BlockSpec((D, D), lambda i: (0, 0)),
        pl.BlockSpec((D, D), lambda i: (0, 0)),
        pl.BlockSpec((2, D), lambda i: (0, 0)),
        pl.BlockSpec((1, D), lambda i: (0, 0)),
        pl.BlockSpec((1, D), lambda i: (0, 0)),
    ],
    out_specs=pl.BlockSpec((_BE, D), lambda i: (i, 0)),
    out_shape=jax.ShapeDtypeStruct((E, D), _f32),
)


def _ec_jnp(src, dst, p, b0):
    a = p[src][:, :D] + p[dst][:, D:] + b0
    b = p[dst][:, :D] + p[src][:, D:] + b0
    f1 = jax.nn.relu(a)
    f2 = jax.nn.relu(b)
    return 0.5 * (f1 + f2), jnp.sum((f1 - f2) ** 2)


def kernel(node_features, edge_index, angles, gt_edges, actions,
           nc1_W0, nc1_b0, nc1_W1, nc1_b1,
           nc2_W0, nc2_b0, nc2_W1, nc2_b1,
           ec1_W0, ec1_b0, ec1_W1, ec1_b1,
           ec2_W0, ec2_b0, ec2_W1, ec2_b1):
    src = edge_index[0]
    dst = edge_index[1]
    x0 = node_features

    _nc_sc_deg = _make_nc_sc(True)
    _nc_sc = _make_nc_sc(False)
    _ec_sc = _make_ec_sc()

    pt1, pb1 = _node_tc_a(x0, nc1_W0[:D], nc1_W0[D:])
    rows = jnp.arange(N, dtype=jnp.int32)
    dst2d = dst.reshape(E // BN2, BN2)
    s1, degp = _nc_sc_deg(src, dst2d, pt1, pb1, nc1_b0, rows)
    ec1_wc = jnp.concatenate([ec1_W0[:D], ec1_W0[D:]], axis=1)
    x1, p1, pt2, pb2 = _node_tc_b(x0, s1, degp, nc1_W1,
                                  nc1_b1.reshape(1, D), ec1_wc,
                                  nc2_W0[:D], nc2_W0[D:])

    h1, ssq1 = _ec_sc(src, dst, p1, ec1_b0)
    (s2,) = _nc_sc(src, dst2d, pt2, pb2, nc2_b0, rows)
    ec2_wc = jnp.concatenate([ec2_W0[:D], ec2_W0[D:]], axis=1)
    x2, p2 = _node_tc_c(x1, s2, degp, nc2_W1, nc2_b1.reshape(1, D), ec2_wc)

    f_w, f01_w, fb_w = _fold_w(ec1_W1[:D], ec1_W1[D:],
                               ec1_b1.reshape(1, 2 * D), ec2_W1[D:])
    h2, ssq2 = _ec_sc(src, dst, p2, ec2_b0)
    ef2 = _ef2_asm(h2, h1, actions, angles, ec2_W1[:D], f_w, f01_w, fb_w,
                   ec2_b1.reshape(1, D))

    side_loss = (jnp.sum(ssq1) + jnp.sum(ssq2)) / (2.0 * E * D)
    return ef2, side_loss


# speedup vs baseline: 1.0070x; 1.0068x over previous
import functools

import jax
import jax.numpy as jnp
from jax import lax
from jax.experimental import pallas as pl
from jax.experimental.pallas import tpu as pltpu
from jax.experimental.pallas import tpu_sc as plsc

N = 10000
E = 160000
D = 128

L = 16
NC = 2
NS = 16
NW = NC * NS
B = 128
NCHUNK = E // B
JMAX = (NCHUNK + NW - 1) // NW
BN = 64
NCHUNKN = E // BN
JMAXN = (NCHUNKN + NW - 1) // NW
ROWS_T = 624
REM = N - NS * ROWS_T
ZROWS = 128

_f32 = jnp.float32


@functools.cache
def _mesh():
    return plsc.VectorSubcoreMesh(core_axis_name="c", subcore_axis_name="s",
                                  num_cores=NC)


def _zero_vmem_rows(buf, rows, width):
    def body(i, _):
        for l in range(width // L):
            buf[i, pl.ds(l * L, L)] = jnp.zeros((L,), _f32)
        return 0
    lax.fori_loop(0, rows, body, 0)


RCH = 16
BN2 = 40
NCW = E // NW // BN2


@functools.cache
def _make_nc_sc(with_deg):
    dbl = not with_deg
    out_type = [jax.ShapeDtypeStruct((NC, N, D), _f32)]
    scratch = [
        pltpu.VMEM((NW * NCW // NW * BN2,), jnp.int32),
        pltpu.VMEM((NCW, BN2), jnp.int32),
        pltpu.VMEM((BN2, D), _f32),
        pltpu.VMEM((BN2, D), _f32),
        pltpu.VMEM((D,), _f32),
        pltpu.VMEM((RCH,), jnp.int32),
        pltpu.VMEM((RCH, D), _f32),
        pltpu.VMEM_SHARED((N, D), _f32),
        pltpu.SemaphoreType.DMA,
        pltpu.SemaphoreType.DMA,
    ]
    if dbl:
        scratch += [
            pltpu.VMEM((BN2, D), _f32),
            pltpu.VMEM((BN2, D), _f32),
            pltpu.SemaphoreType.DMA,
            pltpu.SemaphoreType.DMA,
        ]
    if with_deg:
        out_type.append(jax.ShapeDtypeStruct((NC, N, L), _f32))
        scratch += [
            pltpu.VMEM((BN2, L), _f32),
            pltpu.VMEM((RCH, L), _f32),
            pltpu.VMEM_SHARED((N, L), _f32),
            pltpu.SemaphoreType.DMA,
        ]

    @functools.partial(
        pl.kernel, mesh=_mesh(), out_type=out_type, scratch_types=scratch,
        compiler_params=pltpu.CompilerParams(use_tc_tiling_on_sc=False))
    def nc_kernel(src_hbm, dst2_hbm, pt_hbm, pb_hbm, b0_hbm, rows_hbm,
                  *rest):
        if with_deg:
            (s_out, deg_out, sidx, didx2, a0, g0, b0buf, ridx,
             stage, s_sh, ss0, sd0,
             ones_v, dstage, deg_sh, sdg) = rest
            sets = ((a0, g0, ss0, sd0),)
        else:
            (s_out, sidx, didx2, a0, g0, b0buf, ridx,
             stage, s_sh, ss0, sd0, a1, g1, ss1, sd1) = rest
            sets = ((a0, g0, ss0, sd0), (a1, g1, ss1, sd1))
        cid = lax.axis_index("c")
        sid = lax.axis_index("s")
        wid = sid * NC + cid
        tbase = sid * ROWS_T
        nedge = NCW * BN2

        def load_ridx(base):
            pltpu.sync_copy(rows_hbm.at[pl.ds(base, RCH)], ridx)

        pltpu.sync_copy(src_hbm.at[pl.ds(wid * nedge, nedge)], sidx)
        pltpu.sync_copy(dst2_hbm.at[pl.ds(wid * NCW, NCW)], didx2)

        _zero_vmem_rows(stage, RCH, D)
        if with_deg:
            _zero_vmem_rows(dstage, RCH, L)

            def ones_body(i, _):
                ones_v[i, :] = jnp.ones((L,), _f32)
                return 0
            lax.fori_loop(0, BN2, ones_body, 0)

        pltpu.sync_copy(b0_hbm, b0buf)
        b0v = [b0buf[pl.ds(l * L, L)] for l in range(D // L)]

        for k in range(ROWS_T // RCH):
            load_ridx(tbase + k * RCH)
            pltpu.sync_copy(stage, s_sh.at[ridx])
            if with_deg:
                pltpu.sync_copy(dstage, deg_sh.at[ridx])

        @pl.when(sid == 0)
        def _():
            load_ridx(NS * ROWS_T)
            pltpu.sync_copy(stage, s_sh.at[ridx])
            if with_deg:
                pltpu.sync_copy(dstage, deg_sh.at[ridx])
        plsc.subcore_barrier()

        def fire(k, s):
            a_, g_, sa_, sg_ = sets[s]
            pltpu.async_copy(pt_hbm.at[sidx.at[pl.ds(k * BN2, BN2)]], a_, sa_)
            pltpu.async_copy(pb_hbm.at[didx2.at[k]], g_, sg_)

        def finish(k, s):
            a_, g_, sa_, sg_ = sets[s]
            pltpu.make_async_copy(
                pt_hbm.at[sidx.at[pl.ds(k * BN2, BN2)]], a_, sa_).wait()
            pltpu.make_async_copy(pb_hbm.at[didx2.at[k]], g_, sg_).wait()

            def rbody(i, _):
                for l in range(D // L):
                    sl = pl.ds(l * L, L)
                    a_[i, sl] = jnp.maximum(a_[i, sl] + g_[i, sl] + b0v[l],
                                            0.0)
                return 0
            lax.fori_loop(0, BN2, rbody, 0)
            pltpu.sync_copy(a_, s_sh.at[didx2.at[k]], add=True)
            if with_deg:
                @pl.when(k > 0)
                def _():
                    pltpu.make_async_copy(
                        ones_v, deg_sh.at[didx2.at[k]], sdg).wait()
                pltpu.async_copy(ones_v, deg_sh.at[didx2.at[k]], sdg,
                                 add=True)

        if dbl:
            fire(0, 0)

            def body(t, _):
                k0 = 2 * t
                k1 = k0 + 1

                @pl.when(k1 < NCW)
                def _():
                    fire(k1, 1)
                finish(k0, 0)

                @pl.when(k0 + 2 < NCW)
                def _():
                    fire(k0 + 2, 0)

                @pl.when(k1 < NCW)
                def _():
                    finish(k1, 1)
                return 0
            lax.fori_loop(0, (NCW + 1) // 2, body, 0)
        else:
            def body(k, _):
                fire(k, 0)
                finish(k, 0)
                return 0
            lax.fori_loop(0, NCW, body, 0)
        if with_deg:
            pltpu.make_async_copy(ones_v, deg_sh.at[didx2.at[0]],
                                  sdg).wait()
        plsc.subcore_barrier()

        for k in range(ROWS_T // RCH):
            base = tbase + k * RCH
            load_ridx(base)
            pltpu.sync_copy(s_sh.at[ridx], stage)
            pltpu.sync_copy(stage, s_out.at[cid, pl.ds(base, RCH)])
            if with_deg:
                pltpu.sync_copy(deg_sh.at[ridx], dstage)
                pltpu.sync_copy(dstage, deg_out.at[cid, pl.ds(base, RCH)])

        @pl.when(sid == 0)
        def _():
            load_ridx(NS * ROWS_T)
            pltpu.sync_copy(s_sh.at[ridx], stage)
            pltpu.sync_copy(stage, s_out.at[cid, pl.ds(NS * ROWS_T, REM)])
            if with_deg:
                pltpu.sync_copy(deg_sh.at[ridx], dstage)
                pltpu.sync_copy(dstage,
                                deg_out.at[cid, pl.ds(NS * ROWS_T, REM)])

    return nc_kernel


EB = 40
EW = E // NW
ECHUNKS = EW // EB


@functools.cache
def _make_ec_sc():
    @functools.partial(
        pl.kernel, mesh=_mesh(),
        out_type=[jax.ShapeDtypeStruct((E, D), _f32),
                  jax.ShapeDtypeStruct((NW, 8, L), _f32)],
        scratch_types=[
            pltpu.VMEM((EW,), jnp.int32),
            pltpu.VMEM((EW,), jnp.int32),
            pltpu.VMEM((EB, 2 * D), _f32),
            pltpu.VMEM((EB, 2 * D), _f32),
            pltpu.VMEM((EB, 2 * D), _f32),
            pltpu.VMEM((EB, 2 * D), _f32),
            pltpu.VMEM((5 * EB, D), _f32),
            pltpu.VMEM((D,), _f32),
            pltpu.VMEM((8, L), _f32),
            pltpu.SemaphoreType.DMA,
            pltpu.SemaphoreType.DMA,
            pltpu.SemaphoreType.DMA,
            pltpu.SemaphoreType.DMA,
        ],
        compiler_params=pltpu.CompilerParams(use_tc_tiling_on_sc=False))
    def ec_kernel(src_hbm, dst_hbm, p_hbm, b0_hbm, h_out, ssq_out,
                  sidx, didx, ps0, pd0, ps1, pd1, hbuf, b0buf, ssqacc,
                  ss0, sd0, ss1, sd1):
        cid = lax.axis_index("c")
        sid = lax.axis_index("s")
        wid = sid * NC + cid
        ebase0 = wid * EW

        pltpu.sync_copy(src_hbm.at[pl.ds(ebase0, EW)], sidx)
        pltpu.sync_copy(dst_hbm.at[pl.ds(ebase0, EW)], didx)
        pltpu.sync_copy(b0_hbm, b0buf)
        b0v = [b0buf[pl.ds(l * L, L)] for l in range(D // L)]
        for r in range(8):
            ssqacc[r, :] = jnp.zeros((L,), _f32)

        sets = ((ps0, pd0, ss0, sd0), (ps1, pd1, ss1, sd1))

        def fire(k, s):
            ps_, pd_, ss_, sd_ = sets[s]
            pltpu.async_copy(p_hbm.at[sidx.at[pl.ds(k * EB, EB)]], ps_, ss_)
            pltpu.async_copy(p_hbm.at[didx.at[pl.ds(k * EB, EB)]], pd_, sd_)

        def finish(k, s):
            ps_, pd_, ss_, sd_ = sets[s]
            pltpu.make_async_copy(
                p_hbm.at[sidx.at[pl.ds(k * EB, EB)]], ps_, ss_).wait()
            pltpu.make_async_copy(
                p_hbm.at[didx.at[pl.ds(k * EB, EB)]], pd_, sd_).wait()

            hb = lax.rem(k, 5) * EB

            def row(i, racc):
                for l in range(D // L):
                    sl = pl.ds(l * L, L)
                    sl2 = pl.ds(D + l * L, L)
                    a = ps_[i, sl] + pd_[i, sl2] + b0v[l]
                    b = pd_[i, sl] + ps_[i, sl2] + b0v[l]
                    f1 = jnp.maximum(a, 0.0)
                    f2 = jnp.maximum(b, 0.0)
                    hbuf[hb + i, sl] = 0.5 * (f1 + f2)
                    dd = f1 - f2
                    racc = racc + dd * dd
                return racc
            racc = lax.fori_loop(0, EB, row, jnp.zeros((L,), _f32))
            ssqacc[0, :] = ssqacc[0, :] + racc

            @pl.when(lax.rem(k, 5) == 4)
            def _():
                pltpu.sync_copy(
                    hbuf, h_out.at[pl.ds(ebase0 + (k - 4) * EB, 5 * EB)])

        fire(0, 0)

        def body(t, _):
            k0 = 2 * t
            k1 = k0 + 1

            @pl.when(k1 < ECHUNKS)
            def _():
                fire(k1, 1)
            finish(k0, 0)

            @pl.when(k0 + 2 < ECHUNKS)
            def _():
                fire(k0 + 2, 0)

            @pl.when(k1 < ECHUNKS)
            def _():
                finish(k1, 1)
            return 0
        lax.fori_loop(0, (ECHUNKS + 1) // 2, body, 0)
        pltpu.sync_copy(ssqacc, ssq_out.at[wid])

    return ec_kernel


def _node_tc_body(with_update, nproj, refs):
    if with_update:
        x_ref, sp_ref, degp_ref, w1_ref, b1_ref = refs[:5]
        wrefs = refs[5:5 + nproj]
        outs = refs[5 + nproj:]
        xout = outs[0]
        pouts = outs[1:]
        s = sp_ref[0] + sp_ref[1]
        deg = degp_ref[0][:, 0:1] + degp_ref[1][:, 0:1]
        agg = jnp.dot(s / jnp.maximum(deg, 1.0), w1_ref[...],
                      preferred_element_type=_f32)
        agg = agg + jnp.minimum(deg, 1.0) * b1_ref[...]
        xn = jnp.maximum(x_ref[...] + agg, 0.0)
        xout[...] = xn
    else:
        x_ref = refs[0]
        wrefs = refs[1:1 + nproj]
        pouts = refs[1 + nproj:]
        xn = x_ref[...]
    for w_ref, p_ref in zip(wrefs, pouts):
        p_ref[...] = jnp.dot(xn, w_ref[...], preferred_element_type=_f32)


def _make_node_tc(with_update, proj_widths):
    nproj = len(proj_widths)
    out_shape = []
    if with_update:
        out_shape.append(jax.ShapeDtypeStruct((N, D), _f32))
    out_shape += [jax.ShapeDtypeStruct((N, w), _f32) for w in proj_widths]

    def body(*refs):
        _node_tc_body(with_update, nproj, refs)

    return pl.pallas_call(body, out_shape=out_shape)


_node_tc_a = _make_node_tc(False, [D, D])
_node_tc_b = _make_node_tc(True, [2 * D, D, D])
_node_tc_c = _make_node_tc(True, [2 * D])

_BE = 2000


def _ec1_asm_body(h_ref, act_ref, ang_ref, w1h_ref, w1f_ref, b1_ref, out_ref):
    out_ref[...] = (jnp.dot(h_ref[...], w1h_ref[...],
                            preferred_element_type=_f32)
                    + act_ref[...] * w1f_ref[0:1, :]
                    + ang_ref[...] * w1f_ref[1:2, :]
                    + b1_ref[...])


_ec1_asm = pl.pallas_call(
    _ec1_asm_body,
    grid=(E // _BE,),
    in_specs=[
        pl.BlockSpec((_BE, D), lambda i: (i, 0)),
        pl.BlockSpec((_BE, 1), lambda i: (i, 0)),
        pl.BlockSpec((_BE, 1), lambda i: (i, 0)),
        pl.BlockSpec((D, 2 * D), lambda i: (0, 0)),
        pl.BlockSpec((2, 2 * D), lambda i: (0, 0)),
        pl.BlockSpec((1, 2 * D), lambda i: (0, 0)),
    ],
    out_specs=pl.BlockSpec((_BE, 2 * D), lambda i: (i, 0)),
    out_shape=jax.ShapeDtypeStruct((E, 2 * D), _f32),
)


def _ec2_asm_body(h_ref, ef_ref, w1a_ref, w1b_ref, b1_ref, out_ref):
    out_ref[...] = (jnp.dot(h_ref[...], w1a_ref[...],
                            preferred_element_type=_f32)
                    + jnp.dot(ef_ref[...], w1b_ref[...],
                              preferred_element_type=_f32)
                    + b1_ref[...])


_ec2_asm = pl.pallas_call(
    _ec2_asm_body,
    grid=(E // _BE,),
    in_specs=[
        pl.BlockSpec((_BE, D), lambda i: (i, 0)),
        pl.BlockSpec((_BE, 2 * D), lambda i: (i, 0)),
        pl.BlockSpec((D, D), lambda i: (0, 0)),
        pl.BlockSpec((2 * D, D), lambda i: (0, 0)),
        pl.BlockSpec((1, D), lambda i: (0, 0)),
    ],
    out_specs=pl.BlockSpec((_BE, D), lambda i: (i, 0)),
    out_shape=jax.ShapeDtypeStruct((E, D), _f32),
)


def _ec_jnp(src, dst, p, b0):
    a = p[src][:, :D] + p[dst][:, D:] + b0
    b = p[dst][:, :D] + p[src][:, D:] + b0
    f1 = jax.nn.relu(a)
    f2 = jax.nn.relu(b)
    return 0.5 * (f1 + f2), jnp.sum((f1 - f2) ** 2)


def kernel(node_features, edge_index, angles, gt_edges, actions,
           nc1_W0, nc1_b0, nc1_W1, nc1_b1,
           nc2_W0, nc2_b0, nc2_W1, nc2_b1,
           ec1_W0, ec1_b0, ec1_W1, ec1_b1,
           ec2_W0, ec2_b0, ec2_W1, ec2_b1):
    src = edge_index[0]
    dst = edge_index[1]
    x0 = node_features

    _nc_sc_deg = _make_nc_sc(True)
    _nc_sc = _make_nc_sc(False)
    _ec_sc = _make_ec_sc()

    pt1, pb1 = _node_tc_a(x0, nc1_W0[:D], nc1_W0[D:])
    rows = jnp.arange(N, dtype=jnp.int32)
    dst2d = dst.reshape(E // BN2, BN2)
    s1, degp = _nc_sc_deg(src, dst2d, pt1, pb1, nc1_b0, rows)
    ec1_wc = jnp.concatenate([ec1_W0[:D], ec1_W0[D:]], axis=1)
    x1, p1, pt2, pb2 = _node_tc_b(x0, s1, degp, nc1_W1,
                                  nc1_b1.reshape(1, D), ec1_wc,
                                  nc2_W0[:D], nc2_W0[D:])

    h1, ssq1 = _ec_sc(src, dst, p1, ec1_b0)
    (s2,) = _nc_sc(src, dst2d, pt2, pb2, nc2_b0, rows)
    ec2_wc = jnp.concatenate([ec2_W0[:D], ec2_W0[D:]], axis=1)
    x2, p2 = _node_tc_c(x1, s2, degp, nc2_W1, nc2_b1.reshape(1, D), ec2_wc)

    ef1 = _ec1_asm(h1, actions, angles, ec1_W1[:D], ec1_W1[D:],
                   ec1_b1.reshape(1, 2 * D))
    h2, ssq2 = _ec_sc(src, dst, p2, ec2_b0)
    ef2 = _ec2_asm(h2, ef1, ec2_W1[:D], ec2_W1[D:], ec2_b1.reshape(1, D))

    side_loss = (jnp.sum(ssq1) + jnp.sum(ssq2)) / (2.0 * E * D)
    return ef2, side_loss


# speedup vs baseline: 1.0075x; 1.0005x over previous
import functools

import jax
import jax.numpy as jnp
from jax import lax
from jax.experimental import pallas as pl
from jax.experimental.pallas import tpu as pltpu
from jax.experimental.pallas import tpu_sc as plsc

N = 10000
E = 160000
D = 128

L = 16
NC = 2
NS = 16
NW = NC * NS
ROWS_T = 624
REM = N - NS * ROWS_T

_f32 = jnp.float32


@functools.cache
def _mesh():
    return plsc.VectorSubcoreMesh(core_axis_name="c", subcore_axis_name="s",
                                  num_cores=NC)


def _zero_vmem_rows(buf, rows, width):
    def body(i, _):
        for l in range(width // L):
            buf[i, pl.ds(l * L, L)] = jnp.zeros((L,), _f32)
        return 0
    lax.fori_loop(0, rows, body, 0)


RCH = 16
BN2 = 40
NCW = E // NW // BN2


@functools.cache
def _make_nc_sc(with_deg):
    dbl = not with_deg
    out_type = [jax.ShapeDtypeStruct((NC, N, D), _f32)]
    scratch = [
        pltpu.VMEM((NCW * BN2,), jnp.int32),
        pltpu.VMEM((NCW, BN2), jnp.int32),
        pltpu.VMEM((BN2, D), _f32),
        pltpu.VMEM((BN2, D), _f32),
        pltpu.VMEM((D,), _f32),
        pltpu.VMEM((RCH,), jnp.int32),
        pltpu.VMEM((RCH, D), _f32),
        pltpu.VMEM_SHARED((N, D), _f32),
        pltpu.SemaphoreType.DMA,
        pltpu.SemaphoreType.DMA,
    ]
    if dbl:
        scratch += [
            pltpu.VMEM((BN2, D), _f32),
            pltpu.VMEM((BN2, D), _f32),
            pltpu.SemaphoreType.DMA,
            pltpu.SemaphoreType.DMA,
        ]
    if with_deg:
        out_type.append(jax.ShapeDtypeStruct((NC, N, L), _f32))
        scratch += [
            pltpu.VMEM((BN2, L), _f32),
            pltpu.VMEM((RCH, L), _f32),
            pltpu.VMEM_SHARED((N, L), _f32),
            pltpu.SemaphoreType.DMA,
        ]

    @functools.partial(
        pl.kernel, mesh=_mesh(), out_type=out_type, scratch_types=scratch,
        compiler_params=pltpu.CompilerParams(use_tc_tiling_on_sc=False))
    def nc_kernel(src_hbm, dst2_hbm, pt_hbm, pb_hbm, b0_hbm, rows_hbm,
                  *rest):
        if with_deg:
            (s_out, deg_out, sidx, didx2, a0, g0, b0buf, ridx,
             stage, s_sh, ss0, sd0,
             ones_v, dstage, deg_sh, sdg) = rest
            sets = ((a0, g0, ss0, sd0),)
        else:
            (s_out, sidx, didx2, a0, g0, b0buf, ridx,
             stage, s_sh, ss0, sd0, a1, g1, ss1, sd1) = rest
            sets = ((a0, g0, ss0, sd0), (a1, g1, ss1, sd1))
        cid = lax.axis_index("c")
        sid = lax.axis_index("s")
        wid = sid * NC + cid
        tbase = sid * ROWS_T
        nedge = NCW * BN2

        def load_ridx(base):
            pltpu.sync_copy(rows_hbm.at[pl.ds(base, RCH)], ridx)

        pltpu.sync_copy(src_hbm.at[pl.ds(wid * nedge, nedge)], sidx)
        pltpu.sync_copy(dst2_hbm.at[pl.ds(wid * NCW, NCW)], didx2)

        _zero_vmem_rows(stage, RCH, D)
        if with_deg:
            _zero_vmem_rows(dstage, RCH, L)

            def ones_body(i, _):
                ones_v[i, :] = jnp.ones((L,), _f32)
                return 0
            lax.fori_loop(0, BN2, ones_body, 0)

        pltpu.sync_copy(b0_hbm, b0buf)
        b0v = [b0buf[pl.ds(l * L, L)] for l in range(D // L)]

        for k in range(ROWS_T // RCH):
            load_ridx(tbase + k * RCH)
            pltpu.sync_copy(stage, s_sh.at[ridx])
            if with_deg:
                pltpu.sync_copy(dstage, deg_sh.at[ridx])

        @pl.when(sid == 0)
        def _():
            load_ridx(NS * ROWS_T)
            pltpu.sync_copy(stage, s_sh.at[ridx])
            if with_deg:
                pltpu.sync_copy(dstage, deg_sh.at[ridx])
        plsc.subcore_barrier()

        def fire(k, s):
            a_, g_, sa_, sg_ = sets[s]
            pltpu.async_copy(pt_hbm.at[sidx.at[pl.ds(k * BN2, BN2)]], a_, sa_)
            pltpu.async_copy(pb_hbm.at[didx2.at[k]], g_, sg_)

        def finish(k, s):
            a_, g_, sa_, sg_ = sets[s]
            pltpu.make_async_copy(
                pt_hbm.at[sidx.at[pl.ds(k * BN2, BN2)]], a_, sa_).wait()
            pltpu.make_async_copy(pb_hbm.at[didx2.at[k]], g_, sg_).wait()

            def rbody(i, _):
                for l in range(D // L):
                    sl = pl.ds(l * L, L)
                    a_[i, sl] = jnp.maximum(a_[i, sl] + g_[i, sl] + b0v[l],
                                            0.0)
                return 0
            lax.fori_loop(0, BN2, rbody, 0)
            pltpu.sync_copy(a_, s_sh.at[didx2.at[k]], add=True)
            if with_deg:
                @pl.when(k > 0)
                def _():
                    pltpu.make_async_copy(
                        ones_v, deg_sh.at[didx2.at[k]], sdg).wait()
                pltpu.async_copy(ones_v, deg_sh.at[didx2.at[k]], sdg,
                                 add=True)

        if dbl:
            fire(0, 0)

            def body(t, _):
                k0 = 2 * t
                k1 = k0 + 1

                @pl.when(k1 < NCW)
                def _():
                    fire(k1, 1)
                finish(k0, 0)

                @pl.when(k0 + 2 < NCW)
                def _():
                    fire(k0 + 2, 0)

                @pl.when(k1 < NCW)
                def _():
                    finish(k1, 1)
                return 0
            lax.fori_loop(0, (NCW + 1) // 2, body, 0)
        else:
            def body(k, _):
                fire(k, 0)
                finish(k, 0)
                return 0
            lax.fori_loop(0, NCW, body, 0)
        if with_deg:
            pltpu.make_async_copy(ones_v, deg_sh.at[didx2.at[0]],
                                  sdg).wait()
        plsc.subcore_barrier()

        for k in range(ROWS_T // RCH):
            base = tbase + k * RCH
            load_ridx(base)
            pltpu.sync_copy(s_sh.at[ridx], stage)
            pltpu.sync_copy(stage, s_out.at[cid, pl.ds(base, RCH)])
            if with_deg:
                pltpu.sync_copy(deg_sh.at[ridx], dstage)
                pltpu.sync_copy(dstage, deg_out.at[cid, pl.ds(base, RCH)])

        @pl.when(sid == 0)
        def _():
            load_ridx(NS * ROWS_T)
            pltpu.sync_copy(s_sh.at[ridx], stage)
            pltpu.sync_copy(stage, s_out.at[cid, pl.ds(NS * ROWS_T, REM)])
            if with_deg:
                pltpu.sync_copy(deg_sh.at[ridx], dstage)
                pltpu.sync_copy(dstage,
                                deg_out.at[cid, pl.ds(NS * ROWS_T, REM)])

    return nc_kernel


EB = 40
EW = E // NW
ECHUNKS = EW // EB


@functools.cache
def _make_ec_sc():
    @functools.partial(
        pl.kernel, mesh=_mesh(),
        out_type=[jax.ShapeDtypeStruct((E, D), _f32),
                  jax.ShapeDtypeStruct((NW, 8, L), _f32)],
        scratch_types=[
            pltpu.VMEM((EW,), jnp.int32),
            pltpu.VMEM((EW,), jnp.int32),
            pltpu.VMEM((EB, 2 * D), _f32),
            pltpu.VMEM((EB, 2 * D), _f32),
            pltpu.VMEM((EB, 2 * D), _f32),
            pltpu.VMEM((EB, 2 * D), _f32),
            pltpu.VMEM((5 * EB, D), _f32),
            pltpu.VMEM((D,), _f32),
            pltpu.VMEM((8, L), _f32),
            pltpu.SemaphoreType.DMA,
            pltpu.SemaphoreType.DMA,
            pltpu.SemaphoreType.DMA,
            pltpu.SemaphoreType.DMA,
        ],
        compiler_params=pltpu.CompilerParams(use_tc_tiling_on_sc=False))
    def ec_kernel(src_hbm, dst_hbm, p_hbm, b0_hbm, h_out, ssq_out,
                  sidx, didx, ps0, pd0, ps1, pd1, hbuf, b0buf, ssqacc,
                  ss0, sd0, ss1, sd1):
        cid = lax.axis_index("c")
        sid = lax.axis_index("s")
        wid = sid * NC + cid
        ebase0 = wid * EW

        pltpu.sync_copy(src_hbm.at[pl.ds(ebase0, EW)], sidx)
        pltpu.sync_copy(dst_hbm.at[pl.ds(ebase0, EW)], didx)
        pltpu.sync_copy(b0_hbm, b0buf)
        b0v = [b0buf[pl.ds(l * L, L)] for l in range(D // L)]
        for r in range(8):
            ssqacc[r, :] = jnp.zeros((L,), _f32)

        sets = ((ps0, pd0, ss0, sd0), (ps1, pd1, ss1, sd1))

        def fire(k, s):
            ps_, pd_, ss_, sd_ = sets[s]
            pltpu.async_copy(p_hbm.at[sidx.at[pl.ds(k * EB, EB)]], ps_, ss_)
            pltpu.async_copy(p_hbm.at[didx.at[pl.ds(k * EB, EB)]], pd_, sd_)

        def finish(k, s):
            ps_, pd_, ss_, sd_ = sets[s]
            pltpu.make_async_copy(
                p_hbm.at[sidx.at[pl.ds(k * EB, EB)]], ps_, ss_).wait()
            pltpu.make_async_copy(
                p_hbm.at[didx.at[pl.ds(k * EB, EB)]], pd_, sd_).wait()

            hb = lax.rem(k, 5) * EB

            def row(i, racc):
                for l in range(D // L):
                    sl = pl.ds(l * L, L)
                    sl2 = pl.ds(D + l * L, L)
                    a = ps_[i, sl] + pd_[i, sl2] + b0v[l]
                    b = pd_[i, sl] + ps_[i, sl2] + b0v[l]
                    f1 = jnp.maximum(a, 0.0)
                    f2 = jnp.maximum(b, 0.0)
                    hbuf[hb + i, sl] = 0.5 * (f1 + f2)
                    dd = f1 - f2
                    racc = racc + dd * dd
                return racc
            racc = lax.fori_loop(0, EB, row, jnp.zeros((L,), _f32))
            ssqacc[0, :] = ssqacc[0, :] + racc

            @pl.when(lax.rem(k, 5) == 4)
            def _():
                pltpu.sync_copy(
                    hbuf, h_out.at[pl.ds(ebase0 + (k - 4) * EB, 5 * EB)])

        fire(0, 0)

        def body(t, _):
            k0 = 2 * t
            k1 = k0 + 1

            @pl.when(k1 < ECHUNKS)
            def _():
                fire(k1, 1)
            finish(k0, 0)

            @pl.when(k0 + 2 < ECHUNKS)
            def _():
                fire(k0 + 2, 0)

            @pl.when(k1 < ECHUNKS)
            def _():
                finish(k1, 1)
            return 0
        lax.fori_loop(0, (ECHUNKS + 1) // 2, body, 0)
        pltpu.sync_copy(ssqacc, ssq_out.at[wid])

    return ec_kernel


def _node_tc_body(with_update, nproj, refs):
    if with_update:
        x_ref, sp_ref, degp_ref, w1_ref, b1_ref = refs[:5]
        wrefs = refs[5:5 + nproj]
        outs = refs[5 + nproj:]
        xout = outs[0]
        pouts = outs[1:]
        s = sp_ref[0] + sp_ref[1]
        deg = degp_ref[0][:, 0:1] + degp_ref[1][:, 0:1]
        agg = jnp.dot(s / jnp.maximum(deg, 1.0), w1_ref[...],
                      preferred_element_type=_f32)
        agg = agg + jnp.minimum(deg, 1.0) * b1_ref[...]
        xn = jnp.maximum(x_ref[...] + agg, 0.0)
        xout[...] = xn
    else:
        x_ref = refs[0]
        wrefs = refs[1:1 + nproj]
        pouts = refs[1 + nproj:]
        xn = x_ref[...]
    for w_ref, p_ref in zip(wrefs, pouts):
        p_ref[...] = jnp.dot(xn, w_ref[...], preferred_element_type=_f32)


def _make_node_tc(with_update, proj_widths):
    nproj = len(proj_widths)
    out_shape = []
    if with_update:
        out_shape.append(jax.ShapeDtypeStruct((N, D), _f32))
    out_shape += [jax.ShapeDtypeStruct((N, w), _f32) for w in proj_widths]

    def body(*refs):
        _node_tc_body(with_update, nproj, refs)

    return pl.pallas_call(body, out_shape=out_shape)


_node_tc_a = _make_node_tc(False, [D, D])
_node_tc_b = _make_node_tc(True, [2 * D, D, D])
_node_tc_c = _make_node_tc(True, [2 * D])

_BE = 2000


def _ec1_asm_body(h_ref, act_ref, ang_ref, w1h_ref, w1f_ref, b1_ref, out_ref):
    out_ref[...] = (jnp.dot(h_ref[...], w1h_ref[...],
                            preferred_element_type=_f32)
                    + act_ref[...] * w1f_ref[0:1, :]
                    + ang_ref[...] * w1f_ref[1:2, :]
                    + b1_ref[...])


_ec1_asm = pl.pallas_call(
    _ec1_asm_body,
    grid=(E // _BE,),
    in_specs=[
        pl.BlockSpec((_BE, D), lambda i: (i, 0)),
        pl.BlockSpec((_BE, 1), lambda i: (i, 0)),
        pl.BlockSpec((_BE, 1), lambda i: (i, 0)),
        pl.BlockSpec((D, 2 * D), lambda i: (0, 0)),
        pl.BlockSpec((2, 2 * D), lambda i: (0, 0)),
        pl.BlockSpec((1, 2 * D), lambda i: (0, 0)),
    ],
    out_specs=pl.BlockSpec((_BE, 2 * D), lambda i: (i, 0)),
    out_shape=jax.ShapeDtypeStruct((E, 2 * D), _f32),
)


def _ec2_asm_body(h_ref, ef_ref, w1a_ref, w1b_ref, b1_ref, out_ref):
    out_ref[...] = (jnp.dot(h_ref[...], w1a_ref[...],
                            preferred_element_type=_f32)
                    + jnp.dot(ef_ref[...], w1b_ref[...],
                              preferred_element_type=_f32)
                    + b1_ref[...])


_ec2_asm = pl.pallas_call(
    _ec2_asm_body,
    grid=(E // _BE,),
    in_specs=[
        pl.BlockSpec((_BE, D), lambda i: (i, 0)),
        pl.BlockSpec((_BE, 2 * D), lambda i: (i, 0)),
        pl.BlockSpec((D, D), lambda i: (0, 0)),
        pl.BlockSpec((2 * D, D), lambda i: (0, 0)),
        pl.BlockSpec((1, D), lambda i: (0, 0)),
    ],
    out_specs=pl.BlockSpec((_BE, D), lambda i: (i, 0)),
    out_shape=jax.ShapeDtypeStruct((E, D), _f32),
)


def kernel(node_features, edge_index, angles, gt_edges, actions,
           nc1_W0, nc1_b0, nc1_W1, nc1_b1,
           nc2_W0, nc2_b0, nc2_W1, nc2_b1,
           ec1_W0, ec1_b0, ec1_W1, ec1_b1,
           ec2_W0, ec2_b0, ec2_W1, ec2_b1):
    src = edge_index[0]
    dst = edge_index[1]
    x0 = node_features

    _nc_sc_deg = _make_nc_sc(True)
    _nc_sc = _make_nc_sc(False)
    _ec_sc = _make_ec_sc()

    pt1, pb1 = _node_tc_a(x0, nc1_W0[:D], nc1_W0[D:])
    rows = jnp.arange(N, dtype=jnp.int32)
    dst2d = dst.reshape(E // BN2, BN2)
    s1, degp = _nc_sc_deg(src, dst2d, pt1, pb1, nc1_b0, rows)
    ec1_wc = jnp.concatenate([ec1_W0[:D], ec1_W0[D:]], axis=1)
    x1, p1, pt2, pb2 = _node_tc_b(x0, s1, degp, nc1_W1,
                                  nc1_b1.reshape(1, D), ec1_wc,
                                  nc2_W0[:D], nc2_W0[D:])

    h1, ssq1 = _ec_sc(src, dst, p1, ec1_b0)
    (s2,) = _nc_sc(src, dst2d, pt2, pb2, nc2_b0, rows)
    ec2_wc = jnp.concatenate([ec2_W0[:D], ec2_W0[D:]], axis=1)
    x2, p2 = _node_tc_c(x1, s2, degp, nc2_W1, nc2_b1.reshape(1, D), ec2_wc)

    ef1 = _ec1_asm(h1, actions, angles, ec1_W1[:D], ec1_W1[D:],
                   ec1_b1.reshape(1, 2 * D))
    h2, ssq2 = _ec_sc(src, dst, p2, ec2_b0)
    ef2 = _ec2_asm(h2, ef1, ec2_W1[:D], ec2_W1[D:], ec2_b1.reshape(1, D))

    side_loss = (jnp.sum(ssq1) + jnp.sum(ssq2)) / (2.0 * E * D)
    return ef2, side_loss
